# baseline (speedup 1.0000x reference)
import functools

import jax
import jax.numpy as jnp
from jax import lax
from jax.experimental import pallas as pl
from jax.experimental.pallas import tpu as pltpu
from jax.experimental.pallas import tpu_sc as plsc

NC = 2
NS = 16
NW = NC * NS
L = 16

B = 16384
EMB = 64
BPW = B // NW
CHUNK = 128
NCHUNK = BPW // CHUNK


def _body(u_hbm, v_hbm, user_hbm, item_hbm, out_hbm,
          idx_u, idx_v, rows_u, rows_v, out_v, sem):
    wid = lax.axis_index("s") * NC + lax.axis_index("c")
    base = wid * BPW

    pltpu.sync_copy(u_hbm.at[wid], idx_u)
    pltpu.sync_copy(v_hbm.at[wid], idx_v)

    copies = []
    for c in range(NCHUNK):
        copies.append(pltpu.async_copy(
            user_hbm.at[idx_u.at[c]], rows_u.at[pl.ds(c * CHUNK, CHUNK)], sem))
        copies.append(pltpu.async_copy(
            item_hbm.at[idx_v.at[c]], rows_v.at[pl.ds(c * CHUNK, CHUNK)], sem))
    for cp in copies:
        cp.wait()

    lane = lax.iota(jnp.int32, L)

    def group_body(g, carry):
        rows = g * L + lane
        acc = jnp.zeros((L,), jnp.float32)
        for d in range(EMB):
            col = jnp.full((L,), d, jnp.int32)
            ug = plsc.load_gather(rows_u, [rows, col])
            vg = plsc.load_gather(rows_v, [rows, col])
            acc = acc + ug * vg
        out_v[pl.ds(g * L, L)] = acc
        return carry

    lax.fori_loop(0, BPW // L, group_body, 0)

    pltpu.sync_copy(out_v, out_hbm.at[pl.ds(base, BPW)])


@functools.cache
def _build():
    return pl.kernel(
        _body,
        out_type=jax.ShapeDtypeStruct((B,), jnp.float32),
        mesh=plsc.VectorSubcoreMesh(
            core_axis_name="c", subcore_axis_name="s",
            num_cores=NC, num_subcores=NS),
        scratch_types=[
            pltpu.VMEM((NCHUNK, CHUNK), jnp.int32),
            pltpu.VMEM((NCHUNK, CHUNK), jnp.int32),
            pltpu.VMEM((BPW, EMB), jnp.float32),
            pltpu.VMEM((BPW, EMB), jnp.float32),
            pltpu.VMEM((BPW,), jnp.float32),
            pltpu.SemaphoreType.DMA,
        ],
        compiler_params=pltpu.CompilerParams(
            needs_layout_passes=False, use_tc_tiling_on_sc=False),
    )


def kernel(u, v, user_table, item_table):
    u3 = u.astype(jnp.int32).reshape(NW, NCHUNK, CHUNK)
    v3 = v.astype(jnp.int32).reshape(NW, NCHUNK, CHUNK)
    return _build()(u3, v3, user_table, item_table)


# speedup vs baseline: 6.8758x; 6.8758x over previous
import functools

import jax
import jax.numpy as jnp
from jax import lax
from jax.experimental import pallas as pl
from jax.experimental.pallas import tpu as pltpu
from jax.experimental.pallas import tpu_sc as plsc

NC = 2
NS = 16
NW = NC * NS
L = 16

B = 16384
EMB = 64
BPW = B // NW
G = BPW // L
W = 32
PACK = 128 // W
NP = L // PACK


def _body(u_hbm, v_hbm, tu_hbm, tv_hbm, out_hbm,
          idx_vm, bu, bv, pm, out_v, sem_u, sem_v):
    wid = lax.axis_index("s") * NC + lax.axis_index("c")
    base = wid * BPW

    pltpu.sync_copy(u_hbm.at[pl.ds(base, BPW)], idx_vm.at[0])
    pltpu.sync_copy(v_hbm.at[pl.ds(base, BPW)], idx_vm.at[1])

    lane = lax.iota(jnp.int32, L)
    tvec = lane >> 3
    rvec = lane & 7

    def fire_pack(mu_vec, mv_vec, p):
        for q in range(PACK):
            j = p * PACK + q
            pltpu.async_copy(
                tu_hbm.at[:, :, pl.ds(pl.multiple_of(mu_vec[j], W), W)],
                bu.at[p, :, :, pl.ds(q * W, W)], sem_u.at[p])
            pltpu.async_copy(
                tv_hbm.at[:, :, pl.ds(pl.multiple_of(mv_vec[j], W), W)],
                bv.at[p, :, :, pl.ds(q * W, W)], sem_v.at[p])

    def wait_pack(p):
        pltpu.make_async_copy(tu_hbm.at[:, :, pl.ds(0, 128)],
                              bu.at[p], sem_u.at[p]).wait()
        pltpu.make_async_copy(tv_hbm.at[:, :, pl.ds(0, 128)],
                              bv.at[p], sem_v.at[p]).wait()

    def compute(e, lu, lv, p, q):
        cu = jnp.full((L,), q * W + lu, jnp.int32)
        cv = jnp.full((L,), q * W + lv, jnp.int32)
        sv = jnp.full((L,), p, jnp.int32)
        acc = jnp.zeros((L,), jnp.float32)
        for k in range(EMB // L):
            tv_ = 2 * k + tvec
            du = plsc.load_gather(bu, [sv, tv_, rvec, cu])
            dv = plsc.load_gather(bv, [sv, tv_, rvec, cv])
            acc = acc + du * dv
        plsc.store_scatter(pm, [lane, jnp.full((L,), e, jnp.int32)], acc)

    def group_vecs(g):
        iu = idx_vm[0, pl.ds(g * L, L)]
        iv = idx_vm[1, pl.ds(g * L, L)]
        return iu, iv

    iu0, iv0 = group_vecs(0)
    for p in range(NP):
        fire_pack((iu0 >> 5) << 5, (iv0 >> 5) << 5, p)

    def grp_body(g, carry):
        iu, iv = group_vecs(g)
        lu_vec = iu & (W - 1)
        lv_vec = iv & (W - 1)
        iun, ivn = group_vecs(g + 1)
        mun_vec = (iun >> 5) << 5
        mvn_vec = (ivn >> 5) << 5
        for p in range(NP):
            wait_pack(p)
            for q in range(PACK):
                j = p * PACK + q
                compute(g * L + j, lu_vec[j], lv_vec[j], p, q)
            fire_pack(mun_vec, mvn_vec, p)
        return carry

    lax.fori_loop(0, G - 1, grp_body, 0)

    gl = G - 1
    iu, iv = group_vecs(gl)
    lu_vec = iu & (W - 1)
    lv_vec = iv & (W - 1)
    for p in range(NP):
        wait_pack(p)
        for q in range(PACK):
            j = p * PACK + q
            compute(gl * L + j, lu_vec[j], lv_vec[j], p, q)

    def red_body(g, carry):
        acc = jnp.zeros((L,), jnp.float32)
        for k in range(L):
            acc = acc + pm[k, pl.ds(g * L, L)]
        out_v[pl.ds(g * L, L)] = acc
        return carry

    lax.fori_loop(0, G, red_body, 0)

    pltpu.sync_copy(out_v, out_hbm.at[pl.ds(base, BPW)])


@functools.cache
def _build():
    return pl.kernel(
        _body,
        out_type=jax.ShapeDtypeStruct((B,), jnp.float32),
        mesh=plsc.VectorSubcoreMesh(
            core_axis_name="c", subcore_axis_name="s",
            num_cores=NC, num_subcores=NS),
        scratch_types=[
            pltpu.VMEM((2, BPW), jnp.int32),
            pltpu.VMEM((NP, EMB // 8, 8, 128), jnp.float32),
            pltpu.VMEM((NP, EMB // 8, 8, 128), jnp.float32),
            pltpu.VMEM((L, BPW), jnp.float32),
            pltpu.VMEM((BPW,), jnp.float32),
            pltpu.SemaphoreType.DMA((NP,)),
            pltpu.SemaphoreType.DMA((NP,)),
        ],
        compiler_params=pltpu.CompilerParams(needs_layout_passes=False),
    )


def kernel(u, v, user_table, item_table):
    n = user_table.shape[0]
    tu = user_table.T.reshape(EMB // 8, 8, n)
    tv = item_table.T.reshape(EMB // 8, 8, n)
    return _build()(u.astype(jnp.int32), v.astype(jnp.int32), tu, tv)
